# baseline (speedup 1.0000x reference)
import functools

import jax
import jax.numpy as jnp
from jax import lax
from jax.experimental import pallas as pl
from jax.experimental.pallas import tpu as pltpu
from jax.experimental.pallas import tpu_sc as plsc

_N = 2097152
_C = 32
_NBINS = 15
_LANES = 16
_NW = 32
_ROWS_PER_W = _N // _NW
_CHUNK = 1024
_NCHUNK = _ROWS_PER_W // _CHUNK
_GROUPS = _CHUNK // _LANES

_mesh = plsc.VectorSubcoreMesh(core_axis_name="c", subcore_axis_name="s")


@functools.partial(
    pl.kernel,
    out_type=jax.ShapeDtypeStruct((_NW, 768), jnp.float32),
    mesh=_mesh,
    scratch_types=[
        pltpu.VMEM((_CHUNK // 4, 128), jnp.float32),
        pltpu.VMEM((_CHUNK // 4, 128), jnp.float32),
        pltpu.VMEM((_CHUNK,), jnp.int32),
        pltpu.VMEM((_CHUNK,), jnp.int32),
        pltpu.VMEM((768,), jnp.float32),
        pltpu.SemaphoreType.DMA,
        pltpu.SemaphoreType.DMA,
        pltpu.SemaphoreType.DMA,
        pltpu.SemaphoreType.DMA,
    ],
    compiler_params=pltpu.CompilerParams(
        use_tc_tiling_on_sc=True, needs_layout_passes=False),
)
def _ece_partials(logits_hbm, labels_hbm, out_hbm, lbuf0, lbuf1,
                  labbuf0, labbuf1, acc, ls0, ls1, ts0, ts1):
    lbuf = (lbuf0, lbuf1)
    labbuf = (labbuf0, labbuf1)
    lsem = (ls0, ls1)
    tsem = (ts0, ts1)
    wid = lax.axis_index("s") * 2 + lax.axis_index("c")
    base_row = wid * _ROWS_PER_W

    lane = lax.iota(jnp.int32, _LANES)
    zf = jnp.zeros((_LANES,), jnp.float32)
    onef = jnp.full((_LANES,), 1.0, jnp.float32)
    q0 = jnp.zeros((_LANES,), jnp.int32)
    q1 = jnp.full((_LANES,), 1, jnp.int32)
    q2 = jnp.full((_LANES,), 2, jnp.int32)
    cap = jnp.full((_LANES,), _NBINS - 1, jnp.int32)

    for w in range(768 // _LANES):
        acc[pl.ds(w * _LANES, _LANES)] = zf

    def start(k, b):
        row0 = pl.multiple_of(base_row + k * _CHUNK, _CHUNK)
        q0r = pl.multiple_of((base_row + k * _CHUNK) // 4, _CHUNK // 4)
        pltpu.async_copy(logits_hbm.at[pl.ds(q0r, _CHUNK // 4)], lbuf[b], lsem[b])
        pltpu.async_copy(labels_hbm.at[pl.ds(row0, _CHUNK)], labbuf[b], tsem[b])

    def wait(b):
        pltpu.make_async_copy(
            logits_hbm.at[pl.ds(0, _CHUNK // 4)], lbuf[b], lsem[b]).wait()
        pltpu.make_async_copy(
            labels_hbm.at[pl.ds(0, _CHUNK)], labbuf[b], tsem[b]).wait()

    diagrow, diagcol128, diagcol = [], [], []
    for h in range(2):
        for d in range(_C):
            dc = (lane + d) & (_C - 1)
            fl = (h * _LANES + lane) * _C + dc
            diagrow.append(fl >> 7)
            diagcol128.append(fl & 127)
            diagcol.append(dc)

    def compute(b):
        buf = lbuf[b]

        @pl.loop(0, _GROUPS // 2, unroll=2)
        def _(g):
            g8 = pl.multiple_of(g * 8, 8)
            rowbase = jnp.full((_LANES,), g8, jnp.int32)
            for h in range(2):
                rr = pl.multiple_of(g * (2 * _LANES) + h * _LANES, _LANES)
                cur = [(plsc.load_gather(
                            buf,
                            [rowbase + diagrow[h * _C + d],
                             diagcol128[h * _C + d]]),
                        diagcol[h * _C + d])
                       for d in range(_C)]
                while len(cur) > 1:
                    nxt = []
                    for i in range(0, len(cur), 2):
                        va, ia = cur[i]
                        vb, ib = cur[i + 1]
                        gt = vb > va
                        nxt.append((jnp.maximum(va, vb), jnp.where(gt, ib, ia)))
                    cur = nxt
                m, am = cur[0]
                lab = labbuf[b][pl.ds(rr, _LANES)]
                accv = jnp.where(am == lab, onef, zf)
                conf = onef / (onef + jnp.exp(-m))
                bin_ = jnp.minimum((conf * 15.0).astype(jnp.int32), cap)
                slot = (bin_ << 4) + lane
                plsc.addupdate_scatter(acc, [slot], conf)
                plsc.addupdate_scatter(acc, [slot + jnp.full((_LANES,), 256, jnp.int32)], accv)
                plsc.addupdate_scatter(acc, [slot + jnp.full((_LANES,), 512, jnp.int32)], onef)

    start(0, 0)
    start(1, 1)

    @pl.loop(0, _NCHUNK // 2)
    def _(kk):
        for b in range(2):
            wait(b)
            compute(b)

            @pl.when(kk < _NCHUNK // 2 - 1)
            def _():
                start(kk * 2 + b + 2, b)

    pltpu.sync_copy(acc, out_hbm.at[wid])


_BT = 2048


def _tc_body(lg_ref, lb_ref, out_ref):
    i = pl.program_id(0)
    x = lg_ref[...]
    lbl = lb_ref[0, 0, :]
    m = jnp.max(x, axis=1)
    col = lax.broadcasted_iota(jnp.int32, (_BT, _C), 1)
    am = jnp.min(jnp.where(x == m[:, None], col, _C), axis=1)
    conf = 1.0 / (1.0 + jnp.exp(-m))
    accv = (am == lbl).astype(jnp.float32)
    bi = jnp.minimum((conf * 15.0).astype(jnp.int32), _NBINS - 1)
    onehot = (bi[:, None] == lax.broadcasted_iota(
        jnp.int32, (_BT, _NBINS), 1)).astype(jnp.float32)
    sc_ = jnp.sum(conf[:, None] * onehot, axis=0)
    sa_ = jnp.sum(accv[:, None] * onehot, axis=0)
    cnt = jnp.sum(onehot, axis=0)
    part = jnp.concatenate(
        [sc_, sa_, cnt, jnp.zeros((128 - 3 * _NBINS,), jnp.float32)])

    @pl.when(i == 0)
    def _():
        out_ref[...] = jnp.zeros_like(out_ref)

    out_ref[0, :] += part


def _ece_tc(logits, labels):
    grid = logits.shape[0] // _BT
    out = pl.pallas_call(
        _tc_body,
        grid=(grid,),
        in_specs=[
            pl.BlockSpec((_BT, _C), lambda i: (i, 0)),
            pl.BlockSpec((1, 1, _BT), lambda i: (i, 0, 0)),
        ],
        out_specs=pl.BlockSpec((1, 128), lambda i: (0, 0)),
        out_shape=jax.ShapeDtypeStruct((1, 128), jnp.float32),
    )(logits, labels.reshape(grid, 1, _BT))
    return out[0, :3 * _NBINS].reshape(3, _NBINS)


@jax.jit
def kernel(logits, labels):
    sums = _ece_tc(logits, labels)
    conf_s, acc_s, cnt = sums[0], sums[1], sums[2]
    prop_in_bin = cnt / _N
    safe_cnt = jnp.maximum(cnt, 1.0)
    gap = (conf_s / safe_cnt - acc_s / safe_cnt) * prop_in_bin
    ece = jnp.sum(jnp.where(cnt > 0.0, gap, 0.0))
    return ece.reshape(1)


# speedup vs baseline: 2.1349x; 2.1349x over previous
import functools

import jax
import jax.numpy as jnp
from jax import lax
from jax.experimental import pallas as pl
from jax.experimental.pallas import tpu as pltpu
from jax.experimental.pallas import tpu_sc as plsc

_N = 2097152
_C = 32
_NBINS = 15
_LANES = 16
_NW = 32
_ROWS_PER_W = _N // _NW
_CHUNK = 1024
_NCHUNK = _ROWS_PER_W // _CHUNK
_GROUPS = _CHUNK // _LANES

_mesh = plsc.VectorSubcoreMesh(core_axis_name="c", subcore_axis_name="s")


@functools.partial(
    pl.kernel,
    out_type=jax.ShapeDtypeStruct((_NW, 768), jnp.float32),
    mesh=_mesh,
    scratch_types=[
        pltpu.VMEM((_CHUNK, _C), jnp.float32),
        pltpu.VMEM((_CHUNK, _C), jnp.float32),
        pltpu.VMEM((_CHUNK,), jnp.int32),
        pltpu.VMEM((_CHUNK,), jnp.int32),
        pltpu.VMEM((768,), jnp.float32),
        pltpu.SemaphoreType.DMA,
        pltpu.SemaphoreType.DMA,
        pltpu.SemaphoreType.DMA,
        pltpu.SemaphoreType.DMA,
    ],
    compiler_params=pltpu.CompilerParams(
        use_tc_tiling_on_sc=True, needs_layout_passes=False),
)
def _ece_partials(logits_hbm, labels_hbm, out_hbm, lbuf0, lbuf1,
                  labbuf0, labbuf1, acc, ls0, ls1, ts0, ts1):
    lbuf = (lbuf0, lbuf1)
    labbuf = (labbuf0, labbuf1)
    lsem = (ls0, ls1)
    tsem = (ts0, ts1)
    wid = lax.axis_index("s") * 2 + lax.axis_index("c")
    base_row = wid * _ROWS_PER_W

    lane = lax.iota(jnp.int32, _LANES)
    zf = jnp.zeros((_LANES,), jnp.float32)
    onef = jnp.full((_LANES,), 1.0, jnp.float32)
    q0 = jnp.zeros((_LANES,), jnp.int32)
    q1 = jnp.full((_LANES,), 1, jnp.int32)
    q2 = jnp.full((_LANES,), 2, jnp.int32)
    cap = jnp.full((_LANES,), _NBINS - 1, jnp.int32)

    for w in range(768 // _LANES):
        acc[pl.ds(w * _LANES, _LANES)] = zf

    def start(k, b):
        row0 = pl.multiple_of(base_row + k * _CHUNK, _CHUNK)
        pltpu.async_copy(logits_hbm.at[pl.ds(row0, _CHUNK)], lbuf[b], lsem[b])
        pltpu.async_copy(labels_hbm.at[pl.ds(row0, _CHUNK)], labbuf[b], tsem[b])

    def wait(b):
        pltpu.make_async_copy(
            logits_hbm.at[pl.ds(0, _CHUNK)], lbuf[b], lsem[b]).wait()
        pltpu.make_async_copy(
            labels_hbm.at[pl.ds(0, _CHUNK)], labbuf[b], tsem[b]).wait()

    diagcol = [(lane + d) & (_C - 1) for d in range(_C)]

    def compute(b):
        buf = lbuf[b]

        @pl.loop(0, _GROUPS, unroll=4)
        def _(g):
            rr = pl.multiple_of(g * _LANES, _LANES)
            rows = jnp.full((_LANES,), rr, jnp.int32) + lane
            cur = [(plsc.load_gather(buf, [rows, diagcol[d]]), diagcol[d])
                   for d in range(_C)]
            while len(cur) > 1:
                nxt = []
                for i in range(0, len(cur), 2):
                    va, ia = cur[i]
                    vb, ib = cur[i + 1]
                    gt = vb > va
                    nxt.append((jnp.maximum(va, vb), jnp.where(gt, ib, ia)))
                cur = nxt
            m, am = cur[0]
            lab = labbuf[b][pl.ds(rr, _LANES)]
            accv = jnp.where(am == lab, onef, zf)
            conf = onef / (onef + jnp.exp(-m))
            bin_ = jnp.minimum((conf * 15.0).astype(jnp.int32), cap)
            slot = (bin_ << 4) + lane
            plsc.addupdate_scatter(acc, [slot], conf)
            plsc.addupdate_scatter(acc, [slot + jnp.full((_LANES,), 256, jnp.int32)], accv)
            plsc.addupdate_scatter(acc, [slot + jnp.full((_LANES,), 512, jnp.int32)], onef)

    start(0, 0)
    start(1, 1)

    @pl.loop(0, _NCHUNK // 2)
    def _(kk):
        for b in range(2):
            wait(b)
            compute(b)

            @pl.when(kk < _NCHUNK // 2 - 1)
            def _():
                start(kk * 2 + b + 2, b)

    pltpu.sync_copy(acc, out_hbm.at[wid])


_BT = 8192
_H = _BT // 2


def _tc_half(x, lbl):
    xt = x.T
    m = jnp.max(xt, axis=0)
    maskf = (xt == m[None, :]).astype(jnp.float32)
    colw = lax.broadcasted_iota(jnp.int32, (1, _C), 1).astype(jnp.float32)
    am_f = jax.lax.dot_general(
        colw, maskf, (((1,), (0,)), ((), ())),
        preferred_element_type=jnp.float32)[0]
    conf = 1.0 / (1.0 + jnp.exp(-m))
    accv = (am_f == lbl.astype(jnp.float32)).astype(jnp.float32)
    bi = jnp.minimum((conf * 15.0).astype(jnp.int32), _NBINS - 1)
    onehot = (lax.broadcasted_iota(jnp.int32, (16, _H), 0) ==
              bi[None, :]).astype(jnp.float32)
    feats = jnp.stack([conf, accv, jnp.ones((_H,), jnp.float32)])
    return feats, onehot


def _tc_body(lg_ref, lb_ref, out_ref):
    i = pl.program_id(0)
    lbl = lb_ref[0, 0, :]
    f0, o0 = _tc_half(lg_ref[:_H, :], lbl[:_H])
    f1, o1 = _tc_half(lg_ref[_H:, :], lbl[_H:])
    feats = jnp.concatenate([f0, f1], axis=1)
    onehot = jnp.concatenate([o0, o1], axis=1)
    feats = jnp.concatenate(
        [feats, jnp.zeros((5, _BT), jnp.float32)])
    part = jax.lax.dot_general(
        feats, onehot, (((1,), (1,)), ((), ())),
        preferred_element_type=jnp.float32)

    @pl.when(i == 0)
    def _():
        out_ref[...] = jnp.zeros_like(out_ref)

    out_ref[:, :16] += part


def _ece_tc(logits, labels):
    grid = logits.shape[0] // _BT
    out = pl.pallas_call(
        _tc_body,
        grid=(grid,),
        in_specs=[
            pl.BlockSpec((_BT, _C), lambda i: (i, 0)),
            pl.BlockSpec((1, 1, _BT), lambda i: (i, 0, 0)),
        ],
        out_specs=pl.BlockSpec((8, 128), lambda i: (0, 0)),
        out_shape=jax.ShapeDtypeStruct((8, 128), jnp.float32),
    )(logits, labels.reshape(grid, 1, _BT))
    return out[:3, :_NBINS]


@jax.jit
def kernel(logits, labels):
    sums = _ece_tc(logits, labels)
    conf_s, acc_s, cnt = sums[0], sums[1], sums[2]
    prop_in_bin = cnt / _N
    safe_cnt = jnp.maximum(cnt, 1.0)
    gap = (conf_s / safe_cnt - acc_s / safe_cnt) * prop_in_bin
    ece = jnp.sum(jnp.where(cnt > 0.0, gap, 0.0))
    return ece.reshape(1)
